# Initial kernel scaffold; baseline (speedup 1.0000x reference)
import jax
import jax.numpy as jnp
from jax.experimental import pallas as pl


def kernel(x, edge_index, W1, b1, W2, b2):
    raise NotImplementedError("write your pallas kernel here")


# speedup vs baseline: 282.2221x; 282.2221x over previous
import functools

import jax
import jax.numpy as jnp
from jax import lax
from jax.experimental import pallas as pl
from jax.experimental.pallas import tpu as pltpu
from jax.experimental.pallas import tpu_sc as plsc

N = 100000
E = 6400000
TD = 16

NWORKERS = 32
ZCH = 6272
NF = 16 * ZCH
EROWS = E // 128
NCH = 16
K = NCH * 128
TOTAL_CHUNKS = EROWS // NCH
MAXT = -(-TOTAL_CHUNKS // NWORKERS)

_mesh = plsc.VectorSubcoreMesh(core_axis_name="c", subcore_axis_name="s")


@functools.partial(
    pl.kernel,
    out_type=jax.ShapeDtypeStruct((2, NF), jnp.float32),
    mesh=_mesh,
    compiler_params=pltpu.CompilerParams(
        needs_layout_passes=False,
        use_tc_tiling_on_sc=False,
    ),
    scratch_types=[
        pltpu.VMEM((N,), jnp.float32),
        pltpu.VMEM((NCH, 128), jnp.int32),
        pltpu.VMEM((NCH, 128), jnp.int32),
        pltpu.VMEM((NCH, 128), jnp.float32),
        pltpu.VMEM((ZCH,), jnp.float32),
        pltpu.VMEM_SHARED((NF,), jnp.float32),
        pltpu.SemaphoreType.DMA,
        pltpu.SemaphoreType.DMA,
    ],
)
def _sc_field(x_hbm, e_hbm, out_hbm, xv, row_v, col_v, bond_v, zero_v,
              field_sp, sem_in, sem_sc):
    c = lax.axis_index("c")
    s = lax.axis_index("s")
    wid = s * 2 + c

    cp_x = pltpu.async_copy(x_hbm, xv, sem_in)

    zeros16 = jnp.zeros((16,), jnp.float32)

    def _zbody(i, carry):
        zero_v[pl.ds(i * 16, 16)] = zeros16
        return carry

    lax.fori_loop(0, ZCH // 16, _zbody, 0)
    pltpu.sync_copy(zero_v, field_sp.at[pl.ds(s * ZCH, ZCH)])
    cp_x.wait()
    plsc.subcore_barrier()

    def _chunk(t, carry):
        ch = wid + NWORKERS * t

        @pl.when(ch < TOTAL_CHUNKS)
        def _():
            r0 = ch * NCH
            pltpu.sync_copy(e_hbm.at[0, pl.ds(r0, NCH)], row_v)
            pltpu.sync_copy(e_hbm.at[1, pl.ds(r0, NCH)], col_v)

            def _gather(i, inner):
                for j in range(8):
                    r = row_v[i, pl.ds(j * 16, 16)]
                    cc = col_v[i, pl.ds(j * 16, 16)]
                    xa = plsc.load_gather(xv, [r])
                    xb = plsc.load_gather(xv, [cc])
                    bond_v[i, pl.ds(j * 16, 16)] = xa * xb
                return inner

            lax.fori_loop(0, NCH, _gather, 0)

            descs = [
                pltpu.async_copy(bond_v.at[i], field_sp.at[row_v.at[i]],
                                 sem_sc, add=True)
                for i in range(NCH)
            ]
            for d in descs:
                d.wait()

        return carry

    lax.fori_loop(0, MAXT, _chunk, 0)

    plsc.subcore_barrier()
    pltpu.sync_copy(field_sp.at[pl.ds(s * ZCH, ZCH)],
                    out_hbm.at[c, pl.ds(s * ZCH, ZCH)])


BN = 1024
GRID = NF // BN


def _mlp_body(x_ref, p_ref, w1t_ref, b1_ref, w2t_ref, b2_ref, o_ref):
    xb = x_ref[...]
    p = p_ref[...]
    f = p[0:1, :] + p[1:2, :]
    w1t = w1t_ref[...]
    h = jnp.maximum(w1t[:, 0:1] * xb + w1t[:, 1:2] * f + b1_ref[...], 0.0)
    o16 = jnp.dot(w2t_ref[...], h, preferred_element_type=jnp.float32)
    o16 = jnp.maximum(o16 + b2_ref[...], 0.0)
    o_ref[...] = o16.T


_mlp = pl.pallas_call(
    _mlp_body,
    grid=(GRID,),
    in_specs=[
        pl.BlockSpec((1, BN), lambda i: (0, i)),
        pl.BlockSpec((2, BN), lambda i: (0, i)),
        pl.BlockSpec((TD, 2), lambda i: (0, 0)),
        pl.BlockSpec((TD, 1), lambda i: (0, 0)),
        pl.BlockSpec((TD, TD), lambda i: (0, 0)),
        pl.BlockSpec((TD, 1), lambda i: (0, 0)),
    ],
    out_specs=pl.BlockSpec((BN, TD), lambda i: (i, 0)),
    out_shape=jax.ShapeDtypeStruct((NF, TD), jnp.float32),
)


def kernel(x, edge_index, W1, b1, W2, b2):
    xf = x.reshape((N,))
    e3 = edge_index.astype(jnp.int32).reshape((2, EROWS, 128))
    partial = _sc_field(xf, e3)
    xp = jnp.pad(xf, (0, NF - N)).reshape((1, NF))
    tok = _mlp(xp, partial, W1.T, b1.reshape(TD, 1), W2.T, b2.reshape(TD, 1))
    return tok[:N]


# speedup vs baseline: 385.1275x; 1.3646x over previous
import functools

import jax
import jax.numpy as jnp
from jax import lax
from jax.experimental import pallas as pl
from jax.experimental.pallas import tpu as pltpu
from jax.experimental.pallas import tpu_sc as plsc

N = 100000
E = 6400000
TD = 16

NWORKERS = 32
ZCH = 6272
NF = 16 * ZCH
EROWS = E // 128
NCH = 16
K = NCH * 128
TOTAL_CHUNKS = EROWS // NCH
MAXT = -(-TOTAL_CHUNKS // NWORKERS)
XTAIL = N - 15 * ZCH

_mesh = plsc.VectorSubcoreMesh(core_axis_name="c", subcore_axis_name="s")


@functools.partial(
    pl.kernel,
    out_type=jax.ShapeDtypeStruct((3, NF), jnp.float32),
    mesh=_mesh,
    compiler_params=pltpu.CompilerParams(
        needs_layout_passes=False,
        use_tc_tiling_on_sc=False,
    ),
    scratch_types=[
        pltpu.VMEM((N,), jnp.float32),
        pltpu.VMEM((2, NCH, 128), jnp.int32),
        pltpu.VMEM((2, NCH, 128), jnp.int32),
        pltpu.VMEM((NCH, 128), jnp.float32),
        pltpu.VMEM((NCH, 128), jnp.float32),
        pltpu.VMEM((ZCH,), jnp.float32),
        pltpu.VMEM((128,), jnp.int32),
        pltpu.VMEM_SHARED((NF,), jnp.float32),
        pltpu.SemaphoreType.DMA,
        pltpu.SemaphoreType.DMA,
        pltpu.SemaphoreType.DMA,
        pltpu.SemaphoreType.DMA,
        pltpu.SemaphoreType.DMA,
    ],
)
def _sc_field(x_hbm, e_hbm, out_hbm, xv, idx0, idx1, bond0, bond1, zero_v,
              drain_v, field_sp, sem_x, sem_i0, sem_i1, sem_s0, sem_s1):
    c = lax.axis_index("c")
    s = lax.axis_index("s")
    wid = s * 2 + c

    idx_bufs = (idx0, idx1)
    bond_bufs = (bond0, bond1)
    isems = (sem_i0, sem_i1)
    ssems = (sem_s0, sem_s1)

    cp_x = pltpu.async_copy(x_hbm, xv, sem_x)
    pltpu.async_copy(e_hbm.at[:, pl.ds(wid * NCH, NCH)], idx0, sem_i0)

    zeros16 = jnp.zeros((16,), jnp.float32)

    def _zbody(i, carry):
        zero_v[pl.ds(i * 16, 16)] = zeros16
        return carry

    lax.fori_loop(0, ZCH // 16, _zbody, 0)
    pltpu.sync_copy(zero_v, field_sp.at[pl.ds(s * ZCH, ZCH)])
    cp_x.wait()
    plsc.subcore_barrier()

    def _phase(t, b):
        ch = wid + NWORKERS * t
        valid = ch < TOTAL_CHUNKS
        idx_v = idx_bufs[b]
        bond_v = bond_bufs[b]

        @pl.when(valid)
        def _():
            pltpu.make_async_copy(
                e_hbm.at[:, pl.ds(0, NCH)], idx_v, isems[b]).wait()

            def _g(i, inner):
                for j in range(8):
                    r = idx_v[0, i, pl.ds(j * 16, 16)]
                    cc = idx_v[1, i, pl.ds(j * 16, 16)]
                    xa = plsc.load_gather(xv, [r])
                    xb = plsc.load_gather(xv, [cc])
                    bond_v[i, pl.ds(j * 16, 16)] = xa * xb
                return inner

            lax.fori_loop(0, NCH, _g, 0)

        @pl.when((t > 0) & (ch - NWORKERS < TOTAL_CHUNKS))
        def _():
            for _i in range(NCH):
                pltpu.make_async_copy(
                    e_hbm.at[0, 0], drain_v, ssems[1 - b]).wait()

        @pl.when(ch + NWORKERS < TOTAL_CHUNKS)
        def _():
            r0 = (ch + NWORKERS) * NCH
            pltpu.async_copy(
                e_hbm.at[:, pl.ds(r0, NCH)], idx_bufs[1 - b], isems[1 - b])

        @pl.when(valid)
        def _():
            for i in range(NCH):
                pltpu.async_copy(bond_v.at[i], field_sp.at[idx_v.at[0, i]],
                                 ssems[b], add=True)

    def _pair(tp, carry):
        _phase(tp * 2, 0)
        _phase(tp * 2 + 1, 1)
        return carry

    lax.fori_loop(0, MAXT // 2, _pair, 0)

    @pl.when(wid + NWORKERS * (MAXT - 1) < TOTAL_CHUNKS)
    def _():
        for _i in range(NCH):
            pltpu.make_async_copy(e_hbm.at[0, 0], drain_v, sem_s1).wait()

    @pl.when((c == 0) & (s < 15))
    def _():
        pltpu.sync_copy(xv.at[pl.ds(s * ZCH, ZCH)],
                        out_hbm.at[2, pl.ds(s * ZCH, ZCH)])

    @pl.when((c == 0) & (s == 15))
    def _():
        pltpu.sync_copy(xv.at[pl.ds(15 * ZCH, XTAIL)],
                        out_hbm.at[2, pl.ds(15 * ZCH, XTAIL)])

    plsc.subcore_barrier()
    pltpu.sync_copy(field_sp.at[pl.ds(s * ZCH, ZCH)],
                    out_hbm.at[c, pl.ds(s * ZCH, ZCH)])


BN = 1024
GRID = -(-N // BN)


def _mlp_body(p_ref, w1_ref, b1_ref, w2_ref, b2_ref, o_ref):
    p = p_ref[...]
    feats = jnp.concatenate(
        [p[2:3, :], p[0:1, :] + p[1:2, :]], axis=0)
    h = lax.dot_general(w1_ref[...], feats, (((0,), (0,)), ((), ())),
                        preferred_element_type=jnp.float32)
    h = jnp.maximum(h + b1_ref[...], 0.0)
    o = lax.dot_general(w2_ref[...], h, (((0,), (0,)), ((), ())),
                        preferred_element_type=jnp.float32)
    o = jnp.maximum(o + b2_ref[...], 0.0)
    o_ref[...] = o.T


_mlp = pl.pallas_call(
    _mlp_body,
    grid=(GRID,),
    in_specs=[
        pl.BlockSpec((3, BN), lambda i: (0, i)),
        pl.BlockSpec((2, TD), lambda i: (0, 0)),
        pl.BlockSpec((TD, 1), lambda i: (0, 0)),
        pl.BlockSpec((TD, TD), lambda i: (0, 0)),
        pl.BlockSpec((TD, 1), lambda i: (0, 0)),
    ],
    out_specs=pl.BlockSpec((BN, TD), lambda i: (i, 0)),
    out_shape=jax.ShapeDtypeStruct((N, TD), jnp.float32),
)


def kernel(x, edge_index, W1, b1, W2, b2):
    e3 = edge_index.astype(jnp.int32).reshape((2, EROWS, 128))
    part = _sc_field(x.reshape((N,)), e3)
    return _mlp(part, W1, b1.reshape(TD, 1), W2, b2.reshape(TD, 1))
